# Initial kernel scaffold; baseline (speedup 1.0000x reference)
import jax
import jax.numpy as jnp
from jax.experimental import pallas as pl


def kernel(x, W_q, W_k1, W_k2, W_v1, W_v2, W_o):
    raise NotImplementedError("write your pallas kernel here")


# speedup vs baseline: 1.2310x; 1.2310x over previous
import math
import functools
import jax
import jax.numpy as jnp
from jax.experimental import pallas as pl
from jax.experimental.pallas import tpu as pltpu

N_HEAD = 8
HEAD_DIM = 64
ORDER = 3


def _matmul_kernel(x_ref, w_ref, o_ref):
    o_ref[...] = jax.lax.dot_general(
        x_ref[...].astype(jnp.bfloat16), w_ref[...].astype(jnp.bfloat16),
        (((1,), (0,)), ((), ())),
        preferred_element_type=jnp.float32)


def _pallas_matmul(x, w, bm=256):
    M, K = x.shape
    _, N = w.shape
    return pl.pallas_call(
        _matmul_kernel,
        grid=(M // bm,),
        in_specs=[pl.BlockSpec((bm, K), lambda i: (i, 0)),
                  pl.BlockSpec((K, N), lambda i: (0, 0))],
        out_specs=pl.BlockSpec((bm, N), lambda i: (i, 0)),
        out_shape=jax.ShapeDtypeStruct((M, N), jnp.float32),
    )(x, w)


def _core_kernel(q_ref, gk1_ref, gk2_ref, gv1_ref, gv2_ref, o_ref, *, scale, R):
    q = q_ref[...]
    for r in range(R):
        u = (gk1_ref[r] * q[r][None, :]).astype(jnp.bfloat16)
        a = jax.lax.dot_general(u, gk2_ref[r], (((1,), (1,)), ((), ())),
                                preferred_element_type=jnp.float32) * scale
        m = jnp.max(a, axis=-1, keepdims=True)
        e = jnp.exp(a - m)
        alpha = (e / jnp.sum(e, axis=-1, keepdims=True)).astype(jnp.bfloat16)
        wm = jax.lax.dot_general(alpha, gv2_ref[r], (((1,), (0,)), ((), ())),
                                 preferred_element_type=jnp.float32)
        o_ref[r, :] = jnp.sum(gv1_ref[r] * wm, axis=0)


def _pallas_core(q2, gk1, gk2, gv1, gv2, scale, R=16):
    HT, D = q2.shape
    k = gk1.shape[1]
    kern = functools.partial(_core_kernel, scale=scale, R=R)
    bs_q = pl.BlockSpec((R, D), lambda i: (i, 0))
    bs_g = pl.BlockSpec((R, k, D), lambda i: (i, 0, 0))
    return pl.pallas_call(
        kern,
        grid=(HT // R,),
        in_specs=[bs_q, bs_g, bs_g, bs_g, bs_g],
        out_specs=pl.BlockSpec((R, D), lambda i: (i, 0)),
        out_shape=jax.ShapeDtypeStruct((HT, D), jnp.float32),
    )(q2, gk1, gk2, gv1, gv2)


def kernel(x, W_q, W_k1, W_k2, W_v1, W_v2, W_o):
    B, T, E = x.shape
    H, D = N_HEAD, HEAD_DIM
    k_keep = max(1, math.ceil(T ** (2.0 / ORDER)))
    scale = D ** -0.5

    x2 = x.reshape(T, E)
    Wcat = jnp.concatenate([W_q, W_k1, W_k2, W_v1, W_v2], axis=1)
    proj = _pallas_matmul(x2, Wcat)
    q, k1, k2, v1, v2 = [
        proj[:, i * H * D:(i + 1) * H * D].reshape(T, H, D).transpose(1, 0, 2)
        for i in range(5)
    ]

    t_ar = jnp.arange(T)
    causal = t_ar[:, None] < t_ar[None, :]
    this_k = jnp.minimum(k_keep, t_ar + 1)
    j = jnp.arange(k_keep)
    clip_j = jnp.minimum(j[None, :], this_k[:, None] - 1)

    qb = q.astype(jnp.bfloat16)
    gs = []
    for K_r, V_r in ((k1, v1), (k2, v2)):
        logits = jnp.einsum('htd,hsd->hts', qb, K_r.astype(jnp.bfloat16),
                            preferred_element_type=jnp.float32) * scale
        logits = jnp.where(causal[None], -jnp.inf, logits)
        _, idx = jax.lax.top_k(logits, k_keep)
        idx = jnp.take_along_axis(idx, jnp.broadcast_to(clip_j[None], idx.shape), axis=-1)
        gs.append(jax.vmap(lambda tab, ih: tab[ih])(K_r, idx))
        gs.append(jax.vmap(lambda tab, ih: tab[ih])(V_r, idx))
    gk1, gv1, gk2, gv2 = gs

    out = _pallas_core(q.reshape(H * T, D),
                       gk1.reshape(H * T, k_keep, D),
                       gk2.reshape(H * T, k_keep, D).astype(jnp.bfloat16),
                       gv1.reshape(H * T, k_keep, D),
                       gv2.reshape(H * T, k_keep, D).astype(jnp.bfloat16),
                       scale)
    y = out.reshape(H, T, D).transpose(1, 0, 2).reshape(T, H * D)
    res = _pallas_matmul(y, W_o)
    return res.reshape(B, T, E)


# speedup vs baseline: 6.2455x; 5.0736x over previous
import math
import functools
import jax
import jax.numpy as jnp
from jax.experimental import pallas as pl
from jax.experimental.pallas import tpu as pltpu

N_HEAD = 8
HEAD_DIM = 64
ORDER = 3
K_KEEP = 102
K_PAD = 104


def _matmul_kernel(x_ref, w_ref, o_ref):
    o_ref[...] = jax.lax.dot_general(
        x_ref[...].astype(jnp.bfloat16), w_ref[...].astype(jnp.bfloat16),
        (((1,), (0,)), ((), ())),
        preferred_element_type=jnp.float32)


def _pallas_matmul(x, w, bm=256):
    M, K = x.shape
    _, N = w.shape
    return pl.pallas_call(
        _matmul_kernel,
        grid=(M // bm,),
        in_specs=[pl.BlockSpec((bm, K), lambda i: (i, 0)),
                  pl.BlockSpec((K, N), lambda i: (0, 0))],
        out_specs=pl.BlockSpec((bm, N), lambda i: (i, 0)),
        out_shape=jax.ShapeDtypeStruct((M, N), jnp.float32),
    )(x, w)


def _select_kernel(q_ref, kt_ref, mask_ref, pad_ref):
    T = q_ref.shape[1]
    INT_MIN = jnp.int32(-2147483648)
    lg = jax.lax.dot_general(
        q_ref[0].astype(jnp.bfloat16), kt_ref[0], (((1,), (0,)), ((), ())),
        preferred_element_type=jnp.float32)
    n = jax.lax.bitcast_convert_type(lg, jnp.int32)
    ikey = jnp.where(n < 0, ~n, n ^ INT_MIN)
    ukey = jax.lax.bitcast_convert_type(ikey, jnp.uint32)
    row = jax.lax.broadcasted_iota(jnp.int32, (T, T), 0)
    col = jax.lax.broadcasted_iota(jnp.int32, (T, T), 1)
    ukey = jnp.where(col > row, jnp.uint32(0), ukey)

    th = jnp.zeros((T, 1), jnp.uint32)
    for b in range(31, -1, -1):
        cand = th | jnp.uint32(1 << b)
        cnt = jnp.sum((ukey >= cand).astype(jnp.int32), axis=1, keepdims=True)
        th = jnp.where(cnt >= K_KEEP, cand, th)

    gt = ukey > th
    eq = jnp.logical_and(ukey == th, th > jnp.uint32(0))
    sel = jnp.logical_or(gt, eq)
    mask_ref[0] = sel.astype(jnp.int32)

    valid = ukey > jnp.uint32(0)
    ik = jax.lax.bitcast_convert_type(ukey, jnp.int32) ^ INT_MIN
    vk = jnp.where(valid, ik, jnp.int32(2147483647))
    mn = jnp.min(vk, axis=1, keepdims=True)
    iv = jnp.where(vk == mn, col, jnp.int32(1 << 30))
    pad_ref[0] = jnp.min(iv, axis=1).reshape(1, T)


def _pallas_select(q, kt):
    H, T, D = q.shape
    mask, pad = pl.pallas_call(
        _select_kernel,
        grid=(H,),
        in_specs=[pl.BlockSpec((1, T, D), lambda h: (h, 0, 0)),
                  pl.BlockSpec((1, D, T), lambda h: (h, 0, 0))],
        out_specs=[pl.BlockSpec((1, T, T), lambda h: (h, 0, 0)),
                   pl.BlockSpec((1, 1, T), lambda h: (h, 0, 0))],
        out_shape=[jax.ShapeDtypeStruct((H, T, T), jnp.int32),
                   jax.ShapeDtypeStruct((H, 1, T), jnp.int32)],
    )(q, kt)
    return mask, pad.reshape(H, T)


def _core_kernel(q_ref, g1_ref, g2_ref, o_ref, *, scale, R):
    D = q_ref.shape[1]
    q = q_ref[...]
    for r in range(R):
        g1 = g1_ref[r]
        k1 = g1[:K_KEEP, :D]
        v1 = g1[:K_KEEP, D:]
        g2 = g2_ref[r]
        k2 = g2[:K_KEEP, :D]
        v2 = g2[:K_KEEP, D:]
        u = (k1 * q[r][None, :]).astype(jnp.bfloat16)
        a = jax.lax.dot_general(u, k2, (((1,), (1,)), ((), ())),
                                preferred_element_type=jnp.float32) * scale
        m = jnp.max(a, axis=-1, keepdims=True)
        e = jnp.exp(a - m)
        alpha = (e / jnp.sum(e, axis=-1, keepdims=True)).astype(jnp.bfloat16)
        wm = jax.lax.dot_general(alpha, v2, (((1,), (0,)), ((), ())),
                                 preferred_element_type=jnp.float32)
        o_ref[r, :] = jnp.sum(v1 * wm, axis=0)


def _pallas_core(q2, g1, g2, scale, R=16):
    HT, D = q2.shape
    kern = functools.partial(_core_kernel, scale=scale, R=R)
    return pl.pallas_call(
        kern,
        grid=(HT // R,),
        in_specs=[pl.BlockSpec((R, D), lambda i: (i, 0)),
                  pl.BlockSpec((R, K_PAD, 2 * D), lambda i: (i, 0, 0)),
                  pl.BlockSpec((R, K_PAD, 2 * D), lambda i: (i, 0, 0))],
        out_specs=pl.BlockSpec((R, D), lambda i: (i, 0)),
        out_shape=jax.ShapeDtypeStruct((HT, D), jnp.float32),
    )(q2, g1, g2)


def _compact_gather_xla(mask, pad, table):
    H, T, _ = mask.shape
    s = jnp.arange(T, dtype=jnp.int32)
    key = jnp.where(mask > 0, s[None, None, :], T + s[None, None, :])
    order = jnp.sort(key, axis=-1)[..., :K_PAD]
    cnt = jnp.sum(mask, axis=-1, keepdims=True)
    j = jnp.arange(K_PAD, dtype=jnp.int32)
    base = (jnp.arange(H, dtype=jnp.int32) * T)[:, None, None]
    idx = jnp.where(j[None, None, :] < cnt, order + base, pad[:, :, None])
    return table[idx.reshape(H * T, K_PAD)]


def kernel(x, W_q, W_k1, W_k2, W_v1, W_v2, W_o):
    B, T, E = x.shape
    H, D = N_HEAD, HEAD_DIM
    scale = D ** -0.5

    x2 = x.reshape(T, E)
    Wcat = jnp.concatenate([W_q, W_k1, W_k2, W_v1, W_v2], axis=1)
    proj = _pallas_matmul(x2, Wcat)
    q, k1, k2, v1, v2 = [
        proj[:, i * H * D:(i + 1) * H * D].reshape(T, H, D).transpose(1, 0, 2)
        for i in range(5)
    ]

    mask1, pad1 = _pallas_select(q, k1.transpose(0, 2, 1).astype(jnp.bfloat16))
    mask2, pad2 = _pallas_select(q, k2.transpose(0, 2, 1).astype(jnp.bfloat16))

    tab1 = jnp.concatenate([k1, v1], axis=-1).reshape(H * T, 2 * D)
    tab2 = jnp.concatenate([k2, v2], axis=-1).reshape(H * T, 2 * D).astype(jnp.bfloat16)
    base = (jnp.arange(H, dtype=jnp.int32) * T)[:, None]
    g1 = _compact_gather_xla(mask1, pad1 + base, tab1)
    g2 = _compact_gather_xla(mask2, pad2 + base, tab2)

    out = _pallas_core(q.reshape(H * T, D), g1, g2, scale)
    y = out.reshape(H, T, D).transpose(1, 0, 2).reshape(T, H * D)
    res = _pallas_matmul(y, W_o)
    return res.reshape(B, T, E)


# speedup vs baseline: 14.3550x; 2.2985x over previous
import math
import dataclasses
import functools
import jax
import jax.numpy as jnp
from jax import lax
from jax.experimental import pallas as pl
from jax.experimental.pallas import tpu as pltpu
from jax.experimental.pallas import tpu_sc as plsc

N_HEAD = 8
HEAD_DIM = 64
ORDER = 3
K_KEEP = 102
K_PAD = 104


def _matmul_kernel(x_ref, w_ref, o_ref):
    o_ref[...] = jax.lax.dot_general(
        x_ref[...].astype(jnp.bfloat16), w_ref[...].astype(jnp.bfloat16),
        (((1,), (0,)), ((), ())),
        preferred_element_type=jnp.float32)


def _pallas_matmul(x, w, bm=256):
    M, K = x.shape
    _, N = w.shape
    return pl.pallas_call(
        _matmul_kernel,
        grid=(M // bm,),
        in_specs=[pl.BlockSpec((bm, K), lambda i: (i, 0)),
                  pl.BlockSpec((K, N), lambda i: (0, 0))],
        out_specs=pl.BlockSpec((bm, N), lambda i: (i, 0)),
        out_shape=jax.ShapeDtypeStruct((M, N), jnp.float32),
    )(x, w)


def _select_kernel(q_ref, kt_ref, mask_ref, pad_ref):
    T = q_ref.shape[1]
    INT_MIN = jnp.int32(-2147483648)
    lg = jax.lax.dot_general(
        q_ref[0].astype(jnp.bfloat16), kt_ref[0], (((1,), (0,)), ((), ())),
        preferred_element_type=jnp.float32)
    n = jax.lax.bitcast_convert_type(lg, jnp.int32)
    ikey = jnp.where(n < 0, ~n, n ^ INT_MIN)
    ukey = jax.lax.bitcast_convert_type(ikey, jnp.uint32)
    row = jax.lax.broadcasted_iota(jnp.int32, (T, T), 0)
    col = jax.lax.broadcasted_iota(jnp.int32, (T, T), 1)
    ukey = jnp.where(col > row, jnp.uint32(0), ukey)

    th = jnp.zeros((T, 1), jnp.uint32)
    for b in range(31, -1, -1):
        cand = th | jnp.uint32(1 << b)
        cnt = jnp.sum((ukey >= cand).astype(jnp.int32), axis=1, keepdims=True)
        th = jnp.where(cnt >= K_KEEP, cand, th)

    gt = ukey > th
    eq = jnp.logical_and(ukey == th, th > jnp.uint32(0))
    cnt_gt = jnp.sum(gt.astype(jnp.int32), axis=1, keepdims=True)
    tri = (row <= col).astype(jnp.bfloat16)
    prefix = jax.lax.dot_general(eq.astype(jnp.bfloat16), tri,
                                 (((1,), (0,)), ((), ())),
                                 preferred_element_type=jnp.float32)
    quota = (K_KEEP - cnt_gt).astype(jnp.float32)
    sel = jnp.logical_or(gt, jnp.logical_and(eq, prefix <= quota))
    mask_ref[0] = sel.astype(jnp.int32)

    valid = ukey > jnp.uint32(0)
    ik = jax.lax.bitcast_convert_type(ukey, jnp.int32) ^ INT_MIN
    vk = jnp.where(valid, ik, jnp.int32(2147483647))
    mn = jnp.min(vk, axis=1, keepdims=True)
    iv = jnp.where(vk == mn, col, jnp.int32(1 << 30))
    pad_ref[0] = jnp.min(iv, axis=1).reshape(1, T)


def _pallas_select(q, kt):
    H, T, D = q.shape
    mask, pad = pl.pallas_call(
        _select_kernel,
        grid=(H,),
        in_specs=[pl.BlockSpec((1, T, D), lambda h: (h, 0, 0)),
                  pl.BlockSpec((1, D, T), lambda h: (h, 0, 0))],
        out_specs=[pl.BlockSpec((1, T, T), lambda h: (h, 0, 0)),
                   pl.BlockSpec((1, 1, T), lambda h: (h, 0, 0))],
        out_shape=[jax.ShapeDtypeStruct((H, T, T), jnp.int32),
                   jax.ShapeDtypeStruct((H, 1, T), jnp.int32)],
    )(q, kt)
    return mask, pad.reshape(H, T)


def _core_kernel(q_ref, g1_ref, g2_ref, o_ref, *, scale, R):
    D = q_ref.shape[1]
    q = q_ref[...]
    for r in range(R):
        g1 = g1_ref[r]
        k1 = g1[:K_KEEP, :D]
        v1 = g1[:K_KEEP, D:]
        g2 = g2_ref[r]
        k2 = g2[:K_KEEP, :D].astype(jnp.bfloat16)
        v2 = g2[:K_KEEP, D:].astype(jnp.bfloat16)
        u = (k1 * q[r][None, :]).astype(jnp.bfloat16)
        a = jax.lax.dot_general(u, k2, (((1,), (1,)), ((), ())),
                                preferred_element_type=jnp.float32) * scale
        m = jnp.max(a, axis=-1, keepdims=True)
        e = jnp.exp(a - m)
        alpha = (e / jnp.sum(e, axis=-1, keepdims=True)).astype(jnp.bfloat16)
        wm = jax.lax.dot_general(alpha, v2, (((1,), (0,)), ((), ())),
                                 preferred_element_type=jnp.float32)
        o_ref[r, :] = jnp.sum(v1 * wm, axis=0)


def _pallas_core(q2, g1, g2, scale, R=16):
    HT, D = q2.shape
    kern = functools.partial(_core_kernel, scale=scale, R=R)
    return pl.pallas_call(
        kern,
        grid=(HT // R,),
        in_specs=[pl.BlockSpec((R, D), lambda i: (i, 0)),
                  pl.BlockSpec((R, K_PAD, 2 * D), lambda i: (i, 0, 0)),
                  pl.BlockSpec((R, K_PAD, 2 * D), lambda i: (i, 0, 0))],
        out_specs=pl.BlockSpec((R, D), lambda i: (i, 0)),
        out_shape=jax.ShapeDtypeStruct((HT, D), jnp.float32),
    )(q2, g1, g2)


def _sc_compact_gather(mask, pad_flat, table):
    HT, T = mask.shape
    W = table.shape[1]
    NW = 32
    rows_per = HT // NW
    nchunk = T // 16
    mesh = plsc.VectorSubcoreMesh(core_axis_name="c", subcore_axis_name="s")

    cp = pltpu.CompilerParams()
    if "needs_layout_passes" in pltpu.CompilerParams.__dataclass_fields__:
        cp = dataclasses.replace(cp, needs_layout_passes=False)

    @functools.partial(
        pl.kernel, mesh=mesh, compiler_params=cp,
        out_type=jax.ShapeDtypeStruct((HT, K_PAD, W), table.dtype),
        scratch_types=[
            pltpu.VMEM((T,), jnp.int32),
            pltpu.VMEM((T + 16,), jnp.int32),
            pltpu.VMEM((rows_per,), jnp.int32),
            pltpu.VMEM((K_PAD, W), table.dtype),
            pltpu.SemaphoreType.DMA,
        ],
    )
    def k(mask_hbm, pad_hbm, tab_hbm, out_hbm, mrow_v, idx_v, padv, rows_v, sem):
        wid = lax.axis_index("s") * 2 + lax.axis_index("c")
        base_row = wid * rows_per
        pltpu.sync_copy(pad_hbm.at[pl.ds(base_row, rows_per)], padv)
        iota16 = lax.iota(jnp.int32, 16)

        @pl.loop(0, rows_per)
        def _row(i):
            row = base_row + i
            base_s = (row // T) * T
            pltpu.sync_copy(mask_hbm.at[row], mrow_v)

            def chunk(c, cnt):
                w = mrow_v[pl.ds(c * 16, 16)]
                m = w > 0
                mi = m.astype(jnp.int32)
                svec = iota16 + (base_s + c * 16)
                pos = cnt + plsc.cumsum(mi) - mi
                plsc.store_scatter(idx_v, [pos], svec, mask=m)
                return cnt + jnp.sum(mi)

            cnt = lax.fori_loop(0, nchunk, chunk, jnp.int32(0))

            padvec = plsc.load_gather(padv, [jnp.full((16,), i, jnp.int32)])
            for c in range(K_PAD // 8 // 2 + 1):
                cur = idx_v[pl.ds(c * 16, 16)]
                keep = (iota16 + c * 16) < cnt
                idx_v[pl.ds(c * 16, 16)] = jnp.where(keep, cur, padvec)

            pltpu.async_copy(tab_hbm.at[idx_v.at[pl.ds(0, K_PAD)]],
                             rows_v, sem).wait()
            pltpu.sync_copy(rows_v, out_hbm.at[row])

    return k(mask, pad_flat, table)


def _compact_gather_xla(mask, pad, table):
    H, T, _ = mask.shape
    s = jnp.arange(T, dtype=jnp.int32)
    key = jnp.where(mask > 0, s[None, None, :], T + s[None, None, :])
    order = jnp.sort(key, axis=-1)[..., :K_PAD]
    cnt = jnp.sum(mask, axis=-1, keepdims=True)
    j = jnp.arange(K_PAD, dtype=jnp.int32)
    base = (jnp.arange(H, dtype=jnp.int32) * T)[:, None, None]
    idx = jnp.where(j[None, None, :] < cnt, order + base, pad[:, :, None])
    return table[idx.reshape(H * T, K_PAD)]


def kernel(x, W_q, W_k1, W_k2, W_v1, W_v2, W_o):
    B, T, E = x.shape
    H, D = N_HEAD, HEAD_DIM
    scale = D ** -0.5

    x2 = x.reshape(T, E)
    Wcat = jnp.concatenate([W_q, W_k1, W_k2, W_v1, W_v2], axis=1)
    proj = _pallas_matmul(x2, Wcat)
    q, k1, k2, v1, v2 = [
        proj[:, i * H * D:(i + 1) * H * D].reshape(T, H, D).transpose(1, 0, 2)
        for i in range(5)
    ]

    mask1, pad1 = _pallas_select(q, k1.transpose(0, 2, 1).astype(jnp.bfloat16))
    mask2, pad2 = _pallas_select(q, k2.transpose(0, 2, 1).astype(jnp.bfloat16))

    tab1 = jnp.concatenate([k1, v1], axis=-1).reshape(H * T, 2 * D)
    tab2 = jnp.concatenate([k2, v2], axis=-1).reshape(H * T, 2 * D)
    base = (jnp.arange(H, dtype=jnp.int32) * T)[:, None]
    g1 = _sc_compact_gather(mask1.reshape(H * T, T),
                            (pad1 + base).reshape(H * T), tab1)
    g2 = _sc_compact_gather(mask2.reshape(H * T, T),
                            (pad2 + base).reshape(H * T), tab2)

    out = _pallas_core(q.reshape(H * T, D), g1, g2, scale)
    y = out.reshape(H, T, D).transpose(1, 0, 2).reshape(T, H * D)
    res = _pallas_matmul(y, W_o)
    return res.reshape(B, T, E)


# speedup vs baseline: 16.6238x; 1.1580x over previous
import math
import dataclasses
import functools
import jax
import jax.numpy as jnp
from jax import lax
from jax.experimental import pallas as pl
from jax.experimental.pallas import tpu as pltpu
from jax.experimental.pallas import tpu_sc as plsc

N_HEAD = 8
HEAD_DIM = 64
ORDER = 3
K_KEEP = 102
K_PAD = 104


def _matmul_kernel(x_ref, w_ref, o_ref):
    o_ref[...] = jax.lax.dot_general(
        x_ref[...].astype(jnp.bfloat16), w_ref[...].astype(jnp.bfloat16),
        (((1,), (0,)), ((), ())),
        preferred_element_type=jnp.float32)


def _pallas_matmul(x, w, bm=256):
    M, K = x.shape
    _, N = w.shape
    return pl.pallas_call(
        _matmul_kernel,
        grid=(M // bm,),
        in_specs=[pl.BlockSpec((bm, K), lambda i: (i, 0)),
                  pl.BlockSpec((K, N), lambda i: (0, 0))],
        out_specs=pl.BlockSpec((bm, N), lambda i: (i, 0)),
        out_shape=jax.ShapeDtypeStruct((M, N), jnp.float32),
    )(x, w)


def _select_kernel(q_ref, kt_ref, mask_ref, pad_ref):
    T = q_ref.shape[1]
    INT_MIN = jnp.int32(-2147483648)
    lg = jax.lax.dot_general(
        q_ref[0].astype(jnp.bfloat16), kt_ref[0], (((1,), (0,)), ((), ())),
        preferred_element_type=jnp.float32)
    n = jax.lax.bitcast_convert_type(lg, jnp.int32)
    ikey = jnp.where(n < 0, ~n, n ^ INT_MIN)
    ukey = jax.lax.bitcast_convert_type(ikey, jnp.uint32)
    row = jax.lax.broadcasted_iota(jnp.int32, (T, T), 0)
    col = jax.lax.broadcasted_iota(jnp.int32, (T, T), 1)
    ukey = jnp.where(col > row, jnp.uint32(0), ukey)

    th = jnp.zeros((T, 1), jnp.uint32)
    for b in range(31, -1, -1):
        cand = th | jnp.uint32(1 << b)
        cnt = jnp.sum((ukey >= cand).astype(jnp.int32), axis=1, keepdims=True)
        th = jnp.where(cnt >= K_KEEP, cand, th)

    gt = ukey > th
    eq = jnp.logical_and(ukey == th, th > jnp.uint32(0))
    cnt_gt = jnp.sum(gt.astype(jnp.int32), axis=1, keepdims=True)
    tri = (row <= col).astype(jnp.bfloat16)
    prefix = jax.lax.dot_general(eq.astype(jnp.bfloat16), tri,
                                 (((1,), (0,)), ((), ())),
                                 preferred_element_type=jnp.float32)
    quota = (K_KEEP - cnt_gt).astype(jnp.float32)
    sel = jnp.logical_or(gt, jnp.logical_and(eq, prefix <= quota))
    mask_ref[0] = sel.astype(jnp.int32)

    valid = ukey > jnp.uint32(0)
    ik = jax.lax.bitcast_convert_type(ukey, jnp.int32) ^ INT_MIN
    vk = jnp.where(valid, ik, jnp.int32(2147483647))
    mn = jnp.min(vk, axis=1, keepdims=True)
    iv = jnp.where(vk == mn, col, jnp.int32(1 << 30))
    pad_ref[0] = jnp.min(iv, axis=1).reshape(1, T)


def _pallas_select(q, kt):
    H, T, D = q.shape
    mask, pad = pl.pallas_call(
        _select_kernel,
        grid=(H,),
        in_specs=[pl.BlockSpec((1, T, D), lambda h: (h, 0, 0)),
                  pl.BlockSpec((1, D, T), lambda h: (h, 0, 0))],
        out_specs=[pl.BlockSpec((1, T, T), lambda h: (h, 0, 0)),
                   pl.BlockSpec((1, 1, T), lambda h: (h, 0, 0))],
        out_shape=[jax.ShapeDtypeStruct((H, T, T), jnp.int32),
                   jax.ShapeDtypeStruct((H, 1, T), jnp.int32)],
    )(q, kt)
    return mask, pad.reshape(H, T)


def _core_kernel(q_ref, g1_ref, g2_ref, o_ref, *, scale, R):
    D = q_ref.shape[1]
    q = q_ref[...]
    for r in range(R):
        g1 = g1_ref[r]
        k1 = g1[:K_KEEP, :D]
        v1 = g1[:K_KEEP, D:]
        g2 = g2_ref[r]
        k2 = g2[:K_KEEP, :D].astype(jnp.bfloat16)
        v2 = g2[:K_KEEP, D:].astype(jnp.bfloat16)
        u = (k1 * q[r][None, :]).astype(jnp.bfloat16)
        a = jax.lax.dot_general(u, k2, (((1,), (1,)), ((), ())),
                                preferred_element_type=jnp.float32) * scale
        m = jnp.max(a, axis=-1, keepdims=True)
        e = jnp.exp(a - m)
        alpha = (e / jnp.sum(e, axis=-1, keepdims=True)).astype(jnp.bfloat16)
        wm = jax.lax.dot_general(alpha, v2, (((1,), (0,)), ((), ())),
                                 preferred_element_type=jnp.float32)
        o_ref[r, :] = jnp.sum(v1 * wm, axis=0)


def _pallas_core(q2, g1, g2, scale, R=16):
    HT, D = q2.shape
    kern = functools.partial(_core_kernel, scale=scale, R=R)
    return pl.pallas_call(
        kern,
        grid=(HT // R,),
        in_specs=[pl.BlockSpec((R, D), lambda i: (i, 0)),
                  pl.BlockSpec((R, K_PAD, 2 * D), lambda i: (i, 0, 0)),
                  pl.BlockSpec((R, K_PAD, 2 * D), lambda i: (i, 0, 0))],
        out_specs=pl.BlockSpec((R, D), lambda i: (i, 0)),
        out_shape=jax.ShapeDtypeStruct((HT, D), jnp.float32),
    )(q2, g1, g2)


def _sc_compact_gather(mask, pad_flat, table):
    HT, T = mask.shape
    W = table.shape[1]
    NW = 32
    rows_per = HT // NW
    nchunk = T // 16
    mesh = plsc.VectorSubcoreMesh(core_axis_name="c", subcore_axis_name="s")

    cp = pltpu.CompilerParams()
    if "needs_layout_passes" in pltpu.CompilerParams.__dataclass_fields__:
        cp = dataclasses.replace(cp, needs_layout_passes=False)

    @functools.partial(
        pl.kernel, mesh=mesh, compiler_params=cp,
        out_type=jax.ShapeDtypeStruct((HT, K_PAD, W), table.dtype),
        scratch_types=[
            pltpu.VMEM((T,), jnp.int32),
            pltpu.VMEM((T,), jnp.int32),
            pltpu.VMEM((T + 16,), jnp.int32),
            pltpu.VMEM((T + 16,), jnp.int32),
            pltpu.VMEM((rows_per,), jnp.int32),
            pltpu.VMEM((2, K_PAD, W), table.dtype),
            pltpu.SemaphoreType.DMA((2,)),
            pltpu.SemaphoreType.DMA((2,)),
            pltpu.SemaphoreType.DMA((2,)),
        ],
    )
    def k(mask_hbm, pad_hbm, tab_hbm, out_hbm, mrow_v0, mrow_v1, idx_v0,
          idx_v1, padv, rows_v, msem, gsem, osem):
        mrow_b = (mrow_v0, mrow_v1)
        idx_b = (idx_v0, idx_v1)
        wid = lax.axis_index("s") * 2 + lax.axis_index("c")
        base_row = wid * rows_per
        pltpu.sync_copy(pad_hbm.at[pl.ds(base_row, rows_per)], padv)
        iota16 = lax.iota(jnp.int32, 16)

        def mask_cp(ii, b):
            return pltpu.make_async_copy(mask_hbm.at[base_row + ii],
                                         mrow_b[b], msem.at[b])

        def gather_cp(ii, b):
            return pltpu.make_async_copy(
                tab_hbm.at[idx_b[b].at[pl.ds(0, K_PAD)]],
                rows_v.at[b], gsem.at[b])

        def out_cp(ii, b):
            return pltpu.make_async_copy(rows_v.at[b],
                                         out_hbm.at[base_row + ii], osem.at[b])

        mask_cp(0, 0).start()
        mask_cp(1, 1).start()

        @pl.loop(0, rows_per, step=2)
        def _row(i):
            for b in range(2):
                ii = i + b
                row = base_row + ii
                base_s = (row // T) * T
                mask_cp(ii, b).wait()

                def chunk(c, cnt):
                    w = mrow_b[b][pl.ds(c * 16, 16)]
                    m = w > 0
                    mi = m.astype(jnp.int32)
                    svec = iota16 + (base_s + c * 16)
                    pos = cnt + plsc.cumsum(mi) - mi
                    plsc.store_scatter(idx_b[b], [pos], svec, mask=m)
                    return cnt + jnp.sum(mi)

                cnt = lax.fori_loop(0, nchunk, chunk, jnp.int32(0))

                padvec = plsc.load_gather(padv, [jnp.full((16,), ii, jnp.int32)])
                for c in range(K_PAD // 8 // 2 + 1):
                    cur = idx_b[b][pl.ds(c * 16, 16)]
                    keep = (iota16 + c * 16) < cnt
                    idx_b[b][pl.ds(c * 16, 16)] = jnp.where(keep, cur, padvec)

                @pl.when(ii >= 2)
                def _():
                    out_cp(ii - 2, b).wait()

                gather_cp(ii, b).start()

                @pl.when(ii + 2 < rows_per)
                def _():
                    mask_cp(ii + 2, b).start()

                gather_cp(ii, b).wait()
                out_cp(ii, b).start()

        out_cp(rows_per - 2, 0).wait()
        out_cp(rows_per - 1, 1).wait()

    return k(mask, pad_flat, table)


def _compact_gather_xla(mask, pad, table):
    H, T, _ = mask.shape
    s = jnp.arange(T, dtype=jnp.int32)
    key = jnp.where(mask > 0, s[None, None, :], T + s[None, None, :])
    order = jnp.sort(key, axis=-1)[..., :K_PAD]
    cnt = jnp.sum(mask, axis=-1, keepdims=True)
    j = jnp.arange(K_PAD, dtype=jnp.int32)
    base = (jnp.arange(H, dtype=jnp.int32) * T)[:, None, None]
    idx = jnp.where(j[None, None, :] < cnt, order + base, pad[:, :, None])
    return table[idx.reshape(H * T, K_PAD)]


def kernel(x, W_q, W_k1, W_k2, W_v1, W_v2, W_o):
    B, T, E = x.shape
    H, D = N_HEAD, HEAD_DIM
    scale = D ** -0.5

    x2 = x.reshape(T, E)
    Wcat = jnp.concatenate([W_q, W_k1, W_k2, W_v1, W_v2], axis=1)
    proj = _pallas_matmul(x2, Wcat)
    q, k1, k2, v1, v2 = [
        proj[:, i * H * D:(i + 1) * H * D].reshape(T, H, D).transpose(1, 0, 2)
        for i in range(5)
    ]

    mask1, pad1 = _pallas_select(q, k1.transpose(0, 2, 1).astype(jnp.bfloat16))
    mask2, pad2 = _pallas_select(q, k2.transpose(0, 2, 1).astype(jnp.bfloat16))

    tab1 = jnp.concatenate([k1, v1], axis=-1).reshape(H * T, 2 * D)
    tab2 = jnp.concatenate([k2, v2], axis=-1).reshape(H * T, 2 * D)
    base = (jnp.arange(H, dtype=jnp.int32) * T)[:, None]
    g1 = _sc_compact_gather(mask1.reshape(H * T, T),
                            (pad1 + base).reshape(H * T), tab1)
    g2 = _sc_compact_gather(mask2.reshape(H * T, T),
                            (pad2 + base).reshape(H * T), tab2)

    out = _pallas_core(q.reshape(H * T, D), g1, g2, scale)
    y = out.reshape(H, T, D).transpose(1, 0, 2).reshape(T, H * D)
    res = _pallas_matmul(y, W_o)
    return res.reshape(B, T, E)
